# baseline (speedup 1.0000x reference)
import jax
import jax.numpy as jnp
from jax import lax
from jax.experimental import pallas as pl
from jax.experimental.pallas import tpu as pltpu
from jax.experimental.pallas import tpu_sc as plsc

DEPTH = 1000
T = 16384

_INFO = plsc.get_sparse_core_info()
NC = _INFO.num_cores
NS = _INFO.num_subcores
L = _INFO.num_lanes
NW = NC * NS
TPW = T // NW
ROWS = 32
NB = TPW // ROWS


def _body(x_hbm, out_hbm, idx_v, buf0, buf1, sem0, sem1):
    wid = lax.axis_index("s") * NC + lax.axis_index("c")
    base = wid * TPW

    pltpu.sync_copy(x_hbm.at[0, pl.ds(base, TPW)], idx_v)

    nchunk = (DEPTH + L - 1) // L

    def _zero(i, carry):
        r = i // nchunk
        c = i % nchunk
        start = jnp.minimum(c * L, DEPTH - L)
        z = jnp.zeros((L,), jnp.float32)
        buf0[r, pl.ds(start, L)] = z
        buf1[r, pl.ds(start, L)] = z
        return carry

    lax.fori_loop(0, ROWS * nchunk, _zero, 0)

    iota = lax.iota(jnp.int32, L)
    onesv = jnp.full((L,), 1.0, jnp.float32)
    zerov = jnp.zeros((L,), jnp.float32)
    bufs = (buf0, buf1)
    sems = (sem0, sem1)

    def _scatter(buf, j, val):
        for c in range(ROWS // L):
            rowv = iota + c * L
            colv = idx_v[pl.ds(j * ROWS + c * L, L)]
            plsc.store_scatter(buf, [rowv, colv], val)

    copies = [None] * NB
    for j in range(NB):
        buf = bufs[j % 2]
        sem = sems[j % 2]
        if j >= 2:
            copies[j - 2].wait()
            _scatter(buf, j - 2, zerov)
        _scatter(buf, j, onesv)
        dst = out_hbm.at[0, pl.ds(base + j * ROWS, ROWS), :]
        copies[j] = pltpu.make_async_copy(buf, dst, sem)
        copies[j].start()
    copies[NB - 2].wait()
    copies[NB - 1].wait()


@jax.jit
def _onehot_sc(xf):
    k = pl.kernel(
        _body,
        out_type=jax.ShapeDtypeStruct((1, T, DEPTH), jnp.float32),
        mesh=plsc.VectorSubcoreMesh(core_axis_name="c", subcore_axis_name="s"),
        scratch_types=[
            pltpu.VMEM((TPW,), jnp.int32),
            pltpu.VMEM((ROWS, DEPTH), jnp.float32),
            pltpu.VMEM((ROWS, DEPTH), jnp.float32),
            pltpu.SemaphoreType.DMA,
            pltpu.SemaphoreType.DMA,
        ],
        compiler_params=pltpu.CompilerParams(needs_layout_passes=False),
    )
    return k(xf)


def kernel(x, ones):
    del ones
    return _onehot_sc(x)


# speedup vs baseline: 2.4390x; 2.4390x over previous
import jax
import jax.numpy as jnp
from jax import lax
from jax.experimental import pallas as pl
from jax.experimental.pallas import tpu as pltpu
from jax.experimental.pallas import tpu_sc as plsc

DEPTH = 1000
T = 16384

_INFO = plsc.get_sparse_core_info()
NC = _INFO.num_cores
NS = _INFO.num_subcores
L = _INFO.num_lanes
NW = NC * NS
TPW = T // NW
COLS = 128
NB = TPW // COLS


def _body(x_hbm, out_hbm, idx_v, buf, sem):
    wid = lax.axis_index("s") * NC + lax.axis_index("c")
    base = wid * TPW

    pltpu.sync_copy(x_hbm.at[0, pl.ds(base, TPW)], idx_v)

    def _zero(i, carry):
        z = jnp.zeros((L,), jnp.float32)
        for c in range(COLS // L):
            buf[i, pl.ds(c * L, L)] = z
        return carry

    lax.fori_loop(0, DEPTH, _zero, 0)

    iota = lax.iota(jnp.int32, L)
    onesv = jnp.full((L,), 1.0, jnp.float32)
    zerov = jnp.zeros((L,), jnp.float32)

    def _scatter(j, val):
        for c in range(COLS // L):
            colv = iota + c * L
            xv = idx_v[pl.ds(j * COLS + c * L, L)]
            plsc.store_scatter(buf, [xv, colv], val)

    copies = [None] * NB
    for j in range(NB):
        if j >= 1:
            copies[j - 1].wait()
            _scatter(j - 1, zerov)
        _scatter(j, onesv)
        dst = out_hbm.at[:, pl.ds(base + j * COLS, COLS)]
        copies[j] = pltpu.make_async_copy(buf, dst, sem)
        copies[j].start()
    copies[NB - 1].wait()


@jax.jit
def _onehot_sc(x):
    k = pl.kernel(
        _body,
        out_type=jax.ShapeDtypeStruct((DEPTH, T), jnp.float32),
        mesh=plsc.VectorSubcoreMesh(core_axis_name="c", subcore_axis_name="s"),
        scratch_types=[
            pltpu.VMEM((TPW,), jnp.int32),
            pltpu.VMEM((DEPTH, COLS), jnp.float32),
            pltpu.SemaphoreType.DMA,
        ],
        compiler_params=pltpu.CompilerParams(needs_layout_passes=False),
    )
    out_t = k(x)
    return out_t.T[None, :, :]


def kernel(x, ones):
    del ones
    return _onehot_sc(x)


# speedup vs baseline: 2.4436x; 1.0019x over previous
import jax
import jax.numpy as jnp
from jax import lax
from jax.experimental import pallas as pl
from jax.experimental.pallas import tpu as pltpu
from jax.experimental.pallas import tpu_sc as plsc

DEPTH = 1000
T = 16384

_INFO = plsc.get_sparse_core_info()
NC = _INFO.num_cores
NS = _INFO.num_subcores
L = _INFO.num_lanes
NW = NC * NS
TPW = T // NW
COLS = 128
NB = TPW // COLS


def _body(x_hbm, out_hbm, idx_v, buf, sem):
    wid = lax.axis_index("s") * NC + lax.axis_index("c")
    base = wid * TPW

    pltpu.sync_copy(x_hbm.at[0, pl.ds(base, TPW)], idx_v)

    def _zero(i, carry):
        z = jnp.zeros((L,), jnp.float32)
        for c in range(COLS // L):
            buf[i, pl.ds(c * L, L)] = z
        return carry

    lax.fori_loop(0, DEPTH, _zero, 0)

    iota = lax.iota(jnp.int32, L)
    onesv = jnp.full((L,), 1.0, jnp.float32)
    zerov = jnp.zeros((L,), jnp.float32)

    def _scatter(j, val):
        for c in range(COLS // L):
            colv = iota + c * L
            xv = idx_v[pl.ds(j * COLS + c * L, L)]
            plsc.store_scatter(buf, [xv, colv], val)

    copies = [None] * NB
    for j in range(NB):
        if j >= 1:
            copies[j - 1].wait()
            _scatter(j - 1, zerov)
        _scatter(j, onesv)
        dst = out_hbm.at[:, pl.ds(base + j * COLS, COLS)]
        copies[j] = pltpu.make_async_copy(buf, dst, sem)
        copies[j].start()
    copies[NB - 1].wait()


@jax.jit
def _onehot_sc(x):
    k = pl.kernel(
        _body,
        out_type=jax.ShapeDtypeStruct((DEPTH, T), jnp.float32),
        mesh=plsc.VectorSubcoreMesh(core_axis_name="c", subcore_axis_name="s"),
        scratch_types=[
            pltpu.VMEM((TPW,), jnp.int32),
            pltpu.VMEM((DEPTH, COLS), jnp.float32),
            pltpu.SemaphoreType.DMA,
        ],
        compiler_params=pltpu.CompilerParams(
            needs_layout_passes=False,
            disable_bounds_checks=True,
            disable_semaphore_checks=True,
            skip_device_barrier=True,
        ),
    )
    out_t = k(x)
    return out_t.T[None, :, :]


def kernel(x, ones):
    del ones
    return _onehot_sc(x)
